# Initial kernel scaffold; baseline (speedup 1.0000x reference)
import jax
import jax.numpy as jnp
from jax.experimental import pallas as pl


def kernel(points, W1a, b1a, g1a, beta1a, W1b, b1b, g1b, beta1b, W2a, b2a, g2a, beta2a, W2b, b2b, g2b, beta2b):
    raise NotImplementedError("write your pallas kernel here")


# speedup vs baseline: 12.5713x; 12.5713x over previous
import functools

import jax
import jax.numpy as jnp
from jax import lax
from jax.experimental import pallas as pl
from jax.experimental.pallas import tpu as pltpu

N = 10000
NP = 10240
R = 400
K = 16
T = 20000
NSTEP = 8
EPS = 1e-5
BIG = 1e37


def _elu(x):
    return jnp.where(x > 0, x, jnp.exp(jnp.minimum(x, 0.0)) - 1.0)


def _knn_body(pr_ref, pc_ref, pcr_ref, idx_ref, nb_ref):
    pr = pr_ref[...]
    pc = pc_ref[...]
    sqr = jnp.sum(pr * pr, axis=1, keepdims=True)
    sqc = jnp.sum(pc * pc, axis=0, keepdims=True)
    iotaf = lax.broadcasted_iota(jnp.int32, (1, NP), 1).astype(jnp.float32)
    sqc = sqc + jnp.where(iotaf >= N, BIG, 0.0)
    d = sqr + sqc - 2.0 * jnp.dot(pr, pc, preferred_element_type=jnp.float32)

    lane16 = lax.broadcasted_iota(jnp.int32, (R, K), 1)
    idx_acc = jnp.zeros((R, K), jnp.int32)
    for k in range(K):
        m = jnp.min(d, axis=1, keepdims=True)
        idxf = jnp.min(jnp.where(d == m, iotaf, 1e9), axis=1, keepdims=True)
        oh = (iotaf == idxf).astype(jnp.float32)
        nb_ref[k] = jnp.dot(oh, pcr_ref[...], preferred_element_type=jnp.float32)
        d = d + oh * BIG
        idx_acc = idx_acc + jnp.where(lane16 == k, idxf.astype(jnp.int32), 0)
    idx_ref[...] = idx_acc


def _knn_call(p8, pc8, pcr):
    return pl.pallas_call(
        _knn_body,
        grid=(N // R,),
        in_specs=[
            pl.BlockSpec((R, 8), lambda i: (i, 0)),
            pl.BlockSpec((8, NP), lambda i: (0, 0)),
            pl.BlockSpec((NP, 8), lambda i: (0, 0)),
        ],
        out_specs=[
            pl.BlockSpec((R, K), lambda i: (i, 0)),
            pl.BlockSpec((K, R, 8), lambda i: (0, i, 0)),
        ],
        out_shape=[
            jax.ShapeDtypeStruct((N, K), jnp.int32),
            jax.ShapeDtypeStruct((K, N, 8), jnp.float32),
        ],
        compiler_params=pltpu.CompilerParams(vmem_limit_bytes=100_000_000),
    )(p8, pc8, pcr)


def _c1_body(x_ref, w_ref, bias_ref, y_ref, st_ref):
    step = pl.program_id(0)

    @pl.when(step == 0)
    def _():
        st_ref[...] = jnp.zeros_like(st_ref)

    y = (jnp.dot(x_ref[...], w_ref[...], preferred_element_type=jnp.float32)
         + bias_ref[...])
    y_ref[...] = y
    st_ref[0:1, 0:32] += jnp.sum(y, axis=0, keepdims=True)
    st_ref[1:2, 0:32] += jnp.sum(y * y, axis=0, keepdims=True)


def _c1_call(xin, W16, bias):
    return pl.pallas_call(
        _c1_body,
        grid=(NSTEP,),
        in_specs=[
            pl.BlockSpec((T, 16), lambda i: (i, 0)),
            pl.BlockSpec((16, 32), lambda i: (0, 0)),
            pl.BlockSpec((1, 32), lambda i: (0, 0)),
        ],
        out_specs=[
            pl.BlockSpec((T, 32), lambda i: (i, 0)),
            pl.BlockSpec((8, 128), lambda i: (0, 0)),
        ],
        out_shape=[
            jax.ShapeDtypeStruct((N * K, 32), jnp.float32),
            jax.ShapeDtypeStruct((8, 128), jnp.float32),
        ],
        compiler_params=pltpu.CompilerParams(vmem_limit_bytes=100_000_000),
    )(xin, W16, bias)


def _c2_body(y1_ref, st_ref, g4_ref, g4t_ref, gam_ref, bet_ref, w_ref,
             bias_ref, y2_ref, st2_ref):
    step = pl.program_id(0)

    @pl.when(step == 0)
    def _():
        st2_ref[...] = jnp.zeros_like(st2_ref)

    cnt = 4.0 * N * K
    s = st_ref[0:1, 0:32]
    ss = st_ref[1:2, 0:32]
    gs = jnp.dot(s, g4_ref[...], preferred_element_type=jnp.float32) / cnt
    gss = jnp.dot(ss, g4_ref[...], preferred_element_type=jnp.float32) / cnt
    var_g = gss - gs * gs
    mean_c = jnp.dot(gs, g4t_ref[...], preferred_element_type=jnp.float32)
    var_c = jnp.dot(var_g, g4t_ref[...], preferred_element_type=jnp.float32)
    scale = gam_ref[...] / jnp.sqrt(var_c + EPS)
    shift = bet_ref[...] - mean_c * scale

    a = _elu(y1_ref[...] * scale + shift)
    y2 = jnp.dot(a, w_ref[...], preferred_element_type=jnp.float32) + bias_ref[...]
    y2_ref[...] = y2
    st2_ref[0:1, 0:64] += jnp.sum(y2, axis=0, keepdims=True)
    st2_ref[1:2, 0:64] += jnp.sum(y2 * y2, axis=0, keepdims=True)


def _c2_call(y1, st1, G4, G4T, gam, bet, W, bias):
    return pl.pallas_call(
        _c2_body,
        grid=(NSTEP,),
        in_specs=[
            pl.BlockSpec((T, 32), lambda i: (i, 0)),
            pl.BlockSpec((8, 128), lambda i: (0, 0)),
            pl.BlockSpec((32, 8), lambda i: (0, 0)),
            pl.BlockSpec((8, 32), lambda i: (0, 0)),
            pl.BlockSpec((1, 32), lambda i: (0, 0)),
            pl.BlockSpec((1, 32), lambda i: (0, 0)),
            pl.BlockSpec((32, 64), lambda i: (0, 0)),
            pl.BlockSpec((1, 64), lambda i: (0, 0)),
        ],
        out_specs=[
            pl.BlockSpec((T, 64), lambda i: (i, 0)),
            pl.BlockSpec((8, 128), lambda i: (0, 0)),
        ],
        out_shape=[
            jax.ShapeDtypeStruct((N * K, 64), jnp.float32),
            jax.ShapeDtypeStruct((8, 128), jnp.float32),
        ],
        compiler_params=pltpu.CompilerParams(vmem_limit_bytes=100_000_000),
    )(y1, st1, G4, G4T, gam, bet, W, bias)


def _gn_small(y, gmat, gmat_t, gamma, beta, cnt):
    m_g = jnp.dot(jnp.sum(y, axis=0, keepdims=True), gmat,
                  preferred_element_type=jnp.float32) / cnt
    mean_c = jnp.dot(m_g, gmat_t, preferred_element_type=jnp.float32)
    dz = y - mean_c
    var_g = jnp.dot(jnp.sum(dz * dz, axis=0, keepdims=True), gmat,
                    preferred_element_type=jnp.float32) / cnt
    var_c = jnp.dot(var_g, gmat_t, preferred_element_type=jnp.float32)
    return dz / jnp.sqrt(var_c + EPS) * gamma + beta


def _c3_body(y2_ref, st2_ref, g8_ref, g8t_ref, gam1b_ref, bet1b_ref,
             w2a_ref, b2a_ref, gam2a_ref, bet2a_ref, g16_ref, g16t_ref,
             w2b_ref, b2b_ref, gam2b_ref, bet2b_ref, out_ref, macc):
    step = pl.program_id(0)

    @pl.when(step == 0)
    def _():
        macc[...] = jnp.full_like(macc, -BIG)

    cnt = 8.0 * N * K
    s = st2_ref[0:1, 0:64]
    ss = st2_ref[1:2, 0:64]
    gs = jnp.dot(s, g8_ref[...], preferred_element_type=jnp.float32) / cnt
    gss = jnp.dot(ss, g8_ref[...], preferred_element_type=jnp.float32) / cnt
    var_g = gss - gs * gs
    mean_c = jnp.dot(gs, g8t_ref[...], preferred_element_type=jnp.float32)
    var_c = jnp.dot(var_g, g8t_ref[...], preferred_element_type=jnp.float32)
    scale = gam1b_ref[...] / jnp.sqrt(var_c + EPS)
    shift = bet1b_ref[...] - mean_c * scale

    a = _elu(y2_ref[...] * scale + shift)
    r = jnp.max(a.reshape(T // K, K, 64), axis=0)
    macc[...] = jnp.maximum(macc[...], r)

    @pl.when(step == NSTEP - 1)
    def _():
        z = macc[...]
        y3 = jnp.dot(z, w2a_ref[...], preferred_element_type=jnp.float32) + b2a_ref[...]
        a3 = _elu(_gn_small(y3, g16_ref[...], g16t_ref[...],
                            gam2a_ref[...], bet2a_ref[...], 256.0))
        y4 = jnp.dot(a3, w2b_ref[...], preferred_element_type=jnp.float32) + b2b_ref[...]
        out_ref[...] = _elu(_gn_small(y4, g16_ref[...], g16t_ref[...],
                                      gam2b_ref[...], bet2b_ref[...], 256.0))


def _c3_call(y2, st2, G8, G8T, gam1b, bet1b, W2aT, b2a, gam2a, bet2a,
             G16, G16T, W2bT, b2b, gam2b, bet2b):
    cc = lambda shape: pl.BlockSpec(shape, lambda i: tuple(0 for _ in shape))
    return pl.pallas_call(
        _c3_body,
        grid=(NSTEP,),
        in_specs=[
            pl.BlockSpec((T, 64), lambda i: (i, 0)),
            cc((8, 128)), cc((64, 8)), cc((8, 64)),
            cc((1, 64)), cc((1, 64)),
            cc((64, 128)), cc((1, 128)), cc((1, 128)), cc((1, 128)),
            cc((128, 8)), cc((8, 128)),
            cc((128, 128)), cc((1, 128)), cc((1, 128)), cc((1, 128)),
        ],
        out_specs=pl.BlockSpec((K, 128), lambda i: (0, 0)),
        out_shape=jax.ShapeDtypeStruct((K, 128), jnp.float32),
        scratch_shapes=[pltpu.VMEM((K, 64), jnp.float32)],
        compiler_params=pltpu.CompilerParams(vmem_limit_bytes=100_000_000),
    )(y2, st2, G8, G8T, gam1b, bet1b, W2aT, b2a, gam2a, bet2a,
      G16, G16T, W2bT, b2b, gam2b, bet2b)


def _group_onehot(c, groups):
    g = jnp.arange(c) // (c // groups)
    return (g[:, None] == jnp.arange(groups)[None, :]).astype(jnp.float32)


@jax.jit
def kernel(points, W1a, b1a, g1a, beta1a, W1b, b1b, g1b, beta1b,
           W2a, b2a, g2a, beta2a, W2b, b2b, g2b, beta2b):
    p = points[0].T
    p8 = jnp.zeros((N, 8), jnp.float32).at[:, :3].set(p)
    pc8 = jnp.zeros((8, NP), jnp.float32).at[:3, :N].set(p.T)
    pcr = jnp.zeros((NP, 8), jnp.float32).at[:N, :3].set(p)

    idx, nb = _knn_call(p8, pc8, pcr)
    del idx
    nbm = jnp.transpose(nb, (1, 0, 2)).reshape(N * K, 8)
    pts_rep = jnp.repeat(p8, K, axis=0)
    xin = jnp.concatenate([pts_rep, nbm], axis=1)

    W16 = jnp.zeros((16, 32), jnp.float32)
    W16 = W16.at[:3].set((W1a[:, :3] - W1a[:, 3:]).T)
    W16 = W16.at[8:11].set(W1a[:, 3:].T)
    y1, st1 = _c1_call(xin, W16, b1a[None, :])

    G4, G4T = _group_onehot(32, 8), _group_onehot(32, 8).T
    y2, st2 = _c2_call(y1, st1, G4, G4T, g1a[None, :], beta1a[None, :],
                       W1b.T, b1b[None, :])

    G8, G8T = _group_onehot(64, 8), _group_onehot(64, 8).T
    G16, G16T = _group_onehot(128, 8), _group_onehot(128, 8).T
    out = _c3_call(y2, st2, G8, G8T, g1b[None, :], beta1b[None, :],
                   W2a.T, b2a[None, :], g2a[None, :], beta2a[None, :],
                   G16, G16T, W2b.T, b2b[None, :], g2b[None, :], beta2b[None, :])
    return out.T[None]
